# baseline (speedup 1.0000x reference)
import functools

import jax
import jax.numpy as jnp
from jax import lax
from jax.experimental import pallas as pl
from jax.experimental.pallas import tpu as pltpu
from jax.experimental.pallas import tpu_sc as plsc

NC, NS = 2, 16
NW = NC * NS
B, S, E = 4096, 200, 64
V = 1_000_000
BPW = B // NW
S0 = 128
S1 = S - S0
NBUF = 4
LANES = 16


CB = 4096
NBLK = (V + CB - 1) // CB
VL = NBLK * CB


def _xpose_body(in_ref, out_ref):
    t = in_ref[...].T.astype(jnp.bfloat16)
    out_ref[:, :E] = t[: CB // 2]
    out_ref[:, E:] = t[CB // 2:]


def _xpose(tableT):
    return pl.pallas_call(
        _xpose_body,
        grid=(NBLK,),
        in_specs=[pl.BlockSpec((E, CB), lambda i: (0, i))],
        out_specs=pl.BlockSpec((CB // 2, 2 * E), lambda i: (i, 0)),
        out_shape=jax.ShapeDtypeStruct((NBLK * CB // 2, 2 * E), jnp.bfloat16),
    )(tableT)


def _pool_body(x_hbm, table_hbm, out_hbm, idx_v, bufs, rep_v, sems):
    wid = lax.axis_index("s") * NC + lax.axis_index("c")
    base = wid * BPW

    pltpu.sync_copy(x_hbm.at[pl.ds(base, BPW), :], idx_v)

    def start(e, buf, sem):
        pltpu.async_copy(table_hbm.at[idx_v.at[e, pl.ds(0, S0)]],
                         buf.at[pl.ds(0, S0), :], sem)
        pltpu.async_copy(table_hbm.at[idx_v.at[e, pl.ds(S0, S1)]],
                         buf.at[pl.ds(S0, S1), :], sem)

    def wait(e, buf, sem):
        pltpu.make_async_copy(table_hbm.at[idx_v.at[e, pl.ds(0, S0)]],
                              buf.at[pl.ds(0, S0), :], sem).wait()
        pltpu.make_async_copy(table_hbm.at[idx_v.at[e, pl.ds(S0, S1)]],
                              buf.at[pl.ds(S0, S1), :], sem).wait()

    def reduce_into(e, buf):
        def widen_lo(v):
            return lax.bitcast_convert_type(lax.shift_left(v, 16),
                                            jnp.float32)

        def widen_hi(v):
            return lax.bitcast_convert_type(
                jnp.bitwise_and(v, jnp.int32(-65536)), jnp.float32)

        def body(j, acc):
            a0, a1, a2, a3 = acc
            for k in range(4):
                r = j * 4 + k
                v0 = buf[r, pl.ds(0 * LANES, LANES)]
                v1 = buf[r, pl.ds(1 * LANES, LANES)]
                a0 = a0 + widen_lo(v0)
                a1 = a1 + widen_hi(v0)
                a2 = a2 + widen_lo(v1)
                a3 = a3 + widen_hi(v1)
            return a0, a1, a2, a3

        z = jnp.zeros((LANES,), jnp.float32)
        a0, a1, a2, a3 = lax.fori_loop(0, S // 4, body, (z, z, z, z))
        rep_v[e, pl.ds(0 * LANES, LANES)] = a0
        rep_v[e, pl.ds(1 * LANES, LANES)] = a1
        rep_v[e, pl.ds(2 * LANES, LANES)] = a2
        rep_v[e, pl.ds(3 * LANES, LANES)] = a3

    for k in range(NBUF):
        start(k, bufs[k], sems[k])

    def outer(g, carry):
        for k in range(NBUF):
            e = g * NBUF + k
            wait(e, bufs[k], sems[k])
            reduce_into(e, bufs[k])

            @pl.when(g < BPW // NBUF - 1)
            def _():
                start(e + NBUF, bufs[k], sems[k])
        return carry

    lax.fori_loop(0, BPW // NBUF, outer, 0)

    pltpu.sync_copy(rep_v, out_hbm.at[pl.ds(base, BPW), :])


def _pool(x_lin, table_lin):
    def body(x_hbm, table_hbm, out_hbm, idx_v, b0, b1, b2, b3, rep_v,
             s0, s1, s2, s3):
        _pool_body(x_hbm, table_hbm, out_hbm, idx_v,
                   (b0, b1, b2, b3), rep_v, (s0, s1, s2, s3))

    fn = pl.kernel(
        body,
        out_type=jax.ShapeDtypeStruct((B, E), jnp.float32),
        mesh=plsc.VectorSubcoreMesh(core_axis_name="c", subcore_axis_name="s"),
        scratch_types=[
            pltpu.VMEM((BPW, S), jnp.int32),
            pltpu.VMEM((S, E // 2), jnp.int32),
            pltpu.VMEM((S, E // 2), jnp.int32),
            pltpu.VMEM((S, E // 2), jnp.int32),
            pltpu.VMEM((S, E // 2), jnp.int32),
            pltpu.VMEM((BPW, E), jnp.float32),
            pltpu.SemaphoreType.DMA,
            pltpu.SemaphoreType.DMA,
            pltpu.SemaphoreType.DMA,
            pltpu.SemaphoreType.DMA,
        ],
        compiler_params=pltpu.CompilerParams(use_tc_tiling_on_sc=False),
    )
    return fn(x_lin, table_lin)


def _head_body(rep_ref, inv_ref, w1_ref, b1_ref, w2_ref, b2_ref, out_ref):
    rep = rep_ref[...] * inv_ref[...]
    h = jnp.dot(rep, w1_ref[...], preferred_element_type=jnp.float32)
    h = jnp.maximum(h + b1_ref[...], 0.0)
    out_ref[...] = (
        jnp.dot(h, w2_ref[...], preferred_element_type=jnp.float32)
        + b2_ref[...]
    )


def _head(rep, lengths, W1, b1, W2, b2):
    inv = (1.0 / lengths.astype(jnp.float32)).reshape(B, 1)
    bm = 512
    grid = B // bm
    return pl.pallas_call(
        _head_body,
        grid=(grid,),
        in_specs=[
            pl.BlockSpec((bm, E), lambda i: (i, 0)),
            pl.BlockSpec((bm, 1), lambda i: (i, 0)),
            pl.BlockSpec(W1.shape, lambda i: (0, 0)),
            pl.BlockSpec((1, b1.shape[0]), lambda i: (0, 0)),
            pl.BlockSpec(W2.shape, lambda i: (0, 0)),
            pl.BlockSpec((1, b2.shape[0]), lambda i: (0, 0)),
        ],
        out_specs=pl.BlockSpec((bm, b2.shape[0]), lambda i: (i, 0)),
        out_shape=jax.ShapeDtypeStruct((B, b2.shape[0]), jnp.float32),
    )(rep, inv, W1, b1.reshape(1, -1), W2, b2.reshape(1, -1))


@jax.jit
def kernel(x, lengths, table, W1, b1, W2, b2):
    table_bf = _xpose(table.T)
    table_lin = lax.bitcast_convert_type(
        table_bf.reshape(VL, E // 2, 2), jnp.int32)
    xi = x.astype(jnp.int32)
    x2 = ((xi >> 12) << 12) + 2 * (xi & (CB // 2 - 1)) + ((xi >> 11) & 1)
    rep_sum = _pool(x2, table_lin)
    perm = jnp.concatenate([
        jnp.arange(0, E // 2, 2), jnp.arange(1, E // 2, 2),
        jnp.arange(E // 2, E, 2), jnp.arange(E // 2 + 1, E, 2)])
    return _head(rep_sum, lengths, W1[perm, :], b1, W2, b2)


# speedup vs baseline: 85.0874x; 85.0874x over previous
import functools

import jax
import jax.numpy as jnp
from jax import lax
from jax.experimental import pallas as pl
from jax.experimental.pallas import tpu as pltpu
from jax.experimental.pallas import tpu_sc as plsc

NC, NS = 2, 16
NW = NC * NS
B, S, E = 4096, 200, 64
V = 1_000_000
BPW = B // NW
S0 = 128
S1 = S - S0
NBUF = 4
LANES = 16


CB = 4096
NBLK = (V + CB - 1) // CB
VL = NBLK * CB


def _xpose_body(in_ref, out_ref):
    v = lax.bitcast_convert_type(in_ref[...], jnp.int32)
    r = v + jnp.int32(0x7FFF) + ((v >> 16) & jnp.int32(1))
    lo = (r[: E // 2] >> 16) & jnp.int32(0xFFFF)
    hi = r[E // 2:] & jnp.int32(-65536)
    packed = (lo | hi).T
    pf = lax.bitcast_convert_type(packed, jnp.float32)
    q = CB // 4
    out_ref[:, 0 * 32: 1 * 32] = pf[0 * q: 1 * q]
    out_ref[:, 1 * 32: 2 * 32] = pf[1 * q: 2 * q]
    out_ref[:, 2 * 32: 3 * 32] = pf[2 * q: 3 * q]
    out_ref[:, 3 * 32: 4 * 32] = pf[3 * q: 4 * q]


def _xpose(tableT):
    return pl.pallas_call(
        _xpose_body,
        grid=(NBLK,),
        in_specs=[pl.BlockSpec((E, CB), lambda i: (0, i))],
        out_specs=pl.BlockSpec((CB // 4, 2 * E), lambda i: (i, 0)),
        out_shape=jax.ShapeDtypeStruct((NBLK * CB // 4, 2 * E), jnp.float32),
    )(tableT)


def _pool_body(x_hbm, table_hbm, out_hbm, idx_v, bufs, rep_v, sems):
    wid = lax.axis_index("s") * NC + lax.axis_index("c")
    base = wid * BPW

    pltpu.sync_copy(x_hbm.at[pl.ds(base, BPW), :], idx_v)

    def start(e, buf, sem):
        pltpu.async_copy(table_hbm.at[idx_v.at[e, pl.ds(0, S0)]],
                         buf.at[pl.ds(0, S0), :], sem)
        pltpu.async_copy(table_hbm.at[idx_v.at[e, pl.ds(S0, S1)]],
                         buf.at[pl.ds(S0, S1), :], sem)

    def wait(e, buf, sem):
        pltpu.make_async_copy(table_hbm.at[idx_v.at[e, pl.ds(0, S0)]],
                              buf.at[pl.ds(0, S0), :], sem).wait()
        pltpu.make_async_copy(table_hbm.at[idx_v.at[e, pl.ds(S0, S1)]],
                              buf.at[pl.ds(S0, S1), :], sem).wait()

    def reduce_into(e, buf):
        def widen_lo(v):
            return lax.bitcast_convert_type(lax.shift_left(v, 16),
                                            jnp.float32)

        def widen_hi(v):
            return lax.bitcast_convert_type(
                jnp.bitwise_and(v, jnp.int32(-65536)), jnp.float32)

        def body(j, acc):
            a0, a1, a2, a3 = acc
            for k in range(4):
                r = j * 4 + k
                v0 = lax.bitcast_convert_type(
                    buf[r, pl.ds(0 * LANES, LANES)], jnp.int32)
                v1 = lax.bitcast_convert_type(
                    buf[r, pl.ds(1 * LANES, LANES)], jnp.int32)
                a0 = a0 + widen_lo(v0)
                a1 = a1 + widen_lo(v1)
                a2 = a2 + widen_hi(v0)
                a3 = a3 + widen_hi(v1)
            return a0, a1, a2, a3

        z = jnp.zeros((LANES,), jnp.float32)
        a0, a1, a2, a3 = lax.fori_loop(0, S // 4, body, (z, z, z, z))
        rep_v[e, pl.ds(0 * LANES, LANES)] = a0
        rep_v[e, pl.ds(1 * LANES, LANES)] = a1
        rep_v[e, pl.ds(2 * LANES, LANES)] = a2
        rep_v[e, pl.ds(3 * LANES, LANES)] = a3

    for k in range(NBUF):
        start(k, bufs[k], sems[k])

    def outer(g, carry):
        for k in range(NBUF):
            e = g * NBUF + k
            wait(e, bufs[k], sems[k])
            reduce_into(e, bufs[k])

            @pl.when(g < BPW // NBUF - 1)
            def _():
                start(e + NBUF, bufs[k], sems[k])
        return carry

    lax.fori_loop(0, BPW // NBUF, outer, 0)

    pltpu.sync_copy(rep_v, out_hbm.at[pl.ds(base, BPW), :])


def _pool(x_lin, table_lin):
    def body(x_hbm, table_hbm, out_hbm, idx_v, b0, b1, b2, b3, rep_v,
             s0, s1, s2, s3):
        _pool_body(x_hbm, table_hbm, out_hbm, idx_v,
                   (b0, b1, b2, b3), rep_v, (s0, s1, s2, s3))

    fn = pl.kernel(
        body,
        out_type=jax.ShapeDtypeStruct((B, E), jnp.float32),
        mesh=plsc.VectorSubcoreMesh(core_axis_name="c", subcore_axis_name="s"),
        scratch_types=[
            pltpu.VMEM((BPW, S), jnp.int32),
            pltpu.VMEM((S, E // 2), jnp.float32),
            pltpu.VMEM((S, E // 2), jnp.float32),
            pltpu.VMEM((S, E // 2), jnp.float32),
            pltpu.VMEM((S, E // 2), jnp.float32),
            pltpu.VMEM((BPW, E), jnp.float32),
            pltpu.SemaphoreType.DMA,
            pltpu.SemaphoreType.DMA,
            pltpu.SemaphoreType.DMA,
            pltpu.SemaphoreType.DMA,
        ],
        compiler_params=pltpu.CompilerParams(use_tc_tiling_on_sc=False),
    )
    return fn(x_lin, table_lin)


def _head_body(rep_ref, inv_ref, w1_ref, b1_ref, w2_ref, b2_ref, out_ref):
    rep = rep_ref[...] * inv_ref[...]
    h = jnp.dot(rep, w1_ref[...], preferred_element_type=jnp.float32)
    h = jnp.maximum(h + b1_ref[...], 0.0)
    out_ref[...] = (
        jnp.dot(h, w2_ref[...], preferred_element_type=jnp.float32)
        + b2_ref[...]
    )


def _head(rep, lengths, W1, b1, W2, b2):
    inv = (1.0 / lengths.astype(jnp.float32)).reshape(B, 1)
    bm = 512
    grid = B // bm
    return pl.pallas_call(
        _head_body,
        grid=(grid,),
        in_specs=[
            pl.BlockSpec((bm, E), lambda i: (i, 0)),
            pl.BlockSpec((bm, 1), lambda i: (i, 0)),
            pl.BlockSpec(W1.shape, lambda i: (0, 0)),
            pl.BlockSpec((1, b1.shape[0]), lambda i: (0, 0)),
            pl.BlockSpec(W2.shape, lambda i: (0, 0)),
            pl.BlockSpec((1, b2.shape[0]), lambda i: (0, 0)),
        ],
        out_specs=pl.BlockSpec((bm, b2.shape[0]), lambda i: (i, 0)),
        out_shape=jax.ShapeDtypeStruct((B, b2.shape[0]), jnp.float32),
    )(rep, inv, W1, b1.reshape(1, -1), W2, b2.reshape(1, -1))


@jax.jit
def kernel(x, lengths, table, W1, b1, W2, b2):
    table_lin = _xpose(table.T).reshape(VL, E // 2)
    xi = x.astype(jnp.int32)
    x2 = ((xi >> 12) << 12) + 4 * (xi & (CB // 4 - 1)) + ((xi >> 10) & 3)
    rep_sum = _pool(x2, table_lin)
    return _head(rep_sum, lengths, W1, b1, W2, b2)
